# baseline (speedup 1.0000x reference)
import functools

import jax
import jax.numpy as jnp
from jax import lax
from jax.experimental import pallas as pl
from jax.experimental.pallas import tpu as pltpu
from jax.experimental.pallas import tpu_sc as plsc

NC = 2
NS = 16
NW = NC * NS
LAM_U = 0.01


def _sc_gather_dot(item_r, u_flat, table, B, L, D):
    b_per_w = B // NW
    chunk_b = 8
    n_chunks = b_per_w // chunk_b
    rows_per_chunk = chunk_b * L
    splits = ((0, 96), (96, 104))
    n_full = L // 16
    rem = L - n_full * 16
    l_pad = (n_full + 1) * 16

    mesh = plsc.VectorSubcoreMesh(core_axis_name="c", subcore_axis_name="s")

    @functools.partial(
        pl.kernel,
        out_type=(
            jax.ShapeDtypeStruct((B, L), jnp.float32),
            jax.ShapeDtypeStruct((NW, 16), jnp.float32),
        ),
        mesh=mesh,
        scratch_types=[
            pltpu.VMEM((rows_per_chunk,), jnp.int32),
            pltpu.VMEM((rows_per_chunk, D), jnp.float32),
            pltpu.VMEM((chunk_b * D,), jnp.float32),
            pltpu.VMEM((chunk_b, l_pad), jnp.float32),
            pltpu.VMEM((16,), jnp.float32),
            pltpu.SemaphoreType.DMA,
        ],
        compiler_params=pltpu.CompilerParams(
            use_tc_tiling_on_sc=False, needs_layout_passes=False),
    )
    def k(item_ref, u_ref, table_ref, pred_ref, sq_ref,
          idx_v, rows_v, u_v, pred_v, sq_v, sem):
        wid = lax.axis_index("s") * NC + lax.axis_index("c")
        b0w = wid * b_per_w
        lane = lax.iota(jnp.int32, 16)

        def do_group(base, gi, u0, u1, s0, s1, nrows):
            acc = jnp.zeros((16,), jnp.float32)
            for r in range(nrows):
                i = base + gi * 16 + r
                r0 = rows_v[i, pl.ds(0, 16)]
                r1 = rows_v[i, pl.ds(16, 16)]
                p = jnp.sum(r0 * u0 + r1 * u1)
                acc = jnp.where(lane == r, p, acc)
                s0 = s0 + r0 * r0
                s1 = s1 + r1 * r1
            return acc, s0, s1

        def chunk_body(ci, carry):
            sq0, sq1 = carry
            b0 = b0w + ci * chunk_b
            pltpu.sync_copy(item_ref.at[pl.ds(b0 * L, rows_per_chunk)], idx_v)
            pltpu.sync_copy(u_ref.at[pl.ds(b0 * D, chunk_b * D)], u_v)
            copies = [
                pltpu.make_async_copy(
                    table_ref.at[idx_v.at[pl.ds(bb * L + off, sz)]],
                    rows_v.at[pl.ds(bb * L + off, sz)],
                    sem,
                )
                for bb in range(chunk_b)
                for off, sz in splits
            ]
            for c in copies:
                c.start()
            for c in copies:
                c.wait()
            for bb in range(chunk_b):
                u0 = u_v[pl.ds(bb * D, 16)]
                u1 = u_v[pl.ds(bb * D + 16, 16)]
                base = bb * L

                def grp_body(gi, csq, base=base, u0=u0, u1=u1, bb=bb):
                    s0, s1 = csq
                    acc, s0, s1 = do_group(base, gi, u0, u1, s0, s1, 16)
                    pred_v[bb, pl.ds(gi * 16, 16)] = acc
                    return (s0, s1)

                sq0, sq1 = lax.fori_loop(0, n_full, grp_body, (sq0, sq1))
                if rem:
                    acc, sq0, sq1 = do_group(base, n_full, u0, u1,
                                             sq0, sq1, rem)
                    pred_v[bb, pl.ds(n_full * 16, 16)] = acc
            pltpu.sync_copy(pred_v.at[:, pl.ds(0, L)],
                            pred_ref.at[pl.ds(b0, chunk_b)])
            return (sq0, sq1)

        z = jnp.zeros((16,), jnp.float32)
        sq0, sq1 = lax.fori_loop(0, n_chunks, chunk_body, (z, z))
        sq_v[...] = sq0 + sq1
        pltpu.sync_copy(sq_v, sq_ref.at[wid])

    return k(item_r, u_flat, table)


def _tc_loss(pred, labels, mdsk, u2d, sqp):
    B, L = pred.shape
    D = u2d.shape[1]
    blk = 1024
    grid = B // blk

    def body(pred_ref, lab_ref, msk_ref, u_ref, sq_ref, out_ref, acc_ref):
        i = pl.program_id(0)

        @pl.when(i == 0)
        def _():
            acc_ref[0] = 0.0
            acc_ref[1] = 0.0

        x = pred_ref[...]
        t = lab_ref[...]
        m = msk_ref[...]
        bce = jnp.maximum(x, 0.0) - x * t + jnp.log(1.0 + jnp.exp(-jnp.abs(x)))
        acc_ref[0] += jnp.sum(bce * m)
        acc_ref[1] += jnp.sum(u_ref[...] * u_ref[...])

        @pl.when(i == grid - 1)
        def _():
            gsq = jnp.sum(sq_ref[...])
            out_ref[0, 0] = acc_ref[0] + LAM_U * (
                jnp.sqrt(acc_ref[1]) + jnp.sqrt(gsq))

    out = pl.pallas_call(
        body,
        grid=(grid,),
        in_specs=[
            pl.BlockSpec((blk, L), lambda i: (i, 0)),
            pl.BlockSpec((blk, L), lambda i: (i, 0)),
            pl.BlockSpec((blk, L), lambda i: (i, 0)),
            pl.BlockSpec((blk, D), lambda i: (i, 0)),
            pl.BlockSpec(sqp.shape, lambda i: (0, 0)),
        ],
        out_specs=pl.BlockSpec(memory_space=pltpu.SMEM),
        out_shape=jax.ShapeDtypeStruct((1, 1), jnp.float32),
        scratch_shapes=[pltpu.SMEM((2,), jnp.float32)],
    )(pred, labels, mdsk, u2d, sqp)
    return out[0, 0]


def kernel(user_embedding_update, item, labels, mdsk, item_embeddings):
    B, L = item.shape
    D = user_embedding_update.shape[-1]
    u2d = user_embedding_update.reshape(B, D)
    item_f = item.astype(jnp.int32).reshape(B * L)
    u_f = u2d.reshape(B * D)
    P = 2
    Bp = B // P
    preds, sqps = [], []
    for p in range(P):
        pr, sq = _sc_gather_dot(
            lax.dynamic_slice_in_dim(item_f, p * Bp * L, Bp * L),
            lax.dynamic_slice_in_dim(u_f, p * Bp * D, Bp * D),
            item_embeddings, Bp, L, D)
        preds.append(pr)
        sqps.append(sq)
    pred = jnp.concatenate(preds, axis=0)
    sqp = jnp.concatenate(sqps, axis=0)
    return _tc_loss(pred, labels, mdsk, u2d, sqp)


# speedup vs baseline: 1.0828x; 1.0828x over previous
import functools

import jax
import jax.numpy as jnp
from jax import lax
from jax.experimental import pallas as pl
from jax.experimental.pallas import tpu as pltpu
from jax.experimental.pallas import tpu_sc as plsc

NC = 2
NS = 16
NW = NC * NS
LAM_U = 0.01


def _sc_gather_dot(item_r, u_flat, table, B, L, D):
    b_per_w = B // NW
    chunk_b = 8
    n_chunks = b_per_w // chunk_b
    rows_per_chunk = chunk_b * L
    splits = ((0, 96), (96, 104))
    n_full = L // 16
    rem = L - n_full * 16
    l_pad = (n_full + 1) * 16

    mesh = plsc.VectorSubcoreMesh(core_axis_name="c", subcore_axis_name="s")

    @functools.partial(
        pl.kernel,
        out_type=(
            jax.ShapeDtypeStruct((B, L), jnp.float32),
            jax.ShapeDtypeStruct((NW, 16), jnp.float32),
        ),
        mesh=mesh,
        scratch_types=[
            pltpu.VMEM((rows_per_chunk,), jnp.int32),
            pltpu.VMEM((rows_per_chunk, D), jnp.float32),
            pltpu.VMEM((chunk_b * D,), jnp.float32),
            pltpu.VMEM((chunk_b, l_pad), jnp.float32),
            pltpu.VMEM((16,), jnp.float32),
            pltpu.SemaphoreType.DMA,
        ],
        compiler_params=pltpu.CompilerParams(
            use_tc_tiling_on_sc=False, needs_layout_passes=False),
    )
    def k(item_ref, u_ref, table_ref, pred_ref, sq_ref,
          idx_v, rows_v, u_v, pred_v, sq_v, sem):
        wid = lax.axis_index("s") * NC + lax.axis_index("c")
        b0w = wid * b_per_w
        lane = lax.iota(jnp.int32, 16)

        def do_group(base, gi, u0, u1, s0, s1, nrows):
            acc = jnp.zeros((16,), jnp.float32)
            for r in range(nrows):
                i = base + gi * 16 + r
                r0 = rows_v[i, pl.ds(0, 16)]
                r1 = rows_v[i, pl.ds(16, 16)]
                p = jnp.sum(r0 * u0 + r1 * u1)
                acc = jnp.where(lane == r, p, acc)
                s0 = s0 + r0 * r0
                s1 = s1 + r1 * r1
            return acc, s0, s1

        def chunk_body(ci, carry):
            sq0, sq1 = carry
            b0 = b0w + ci * chunk_b
            pltpu.sync_copy(item_ref.at[pl.ds(b0 * L, rows_per_chunk)], idx_v)
            pltpu.sync_copy(u_ref.at[pl.ds(b0 * D, chunk_b * D)], u_v)
            copies = [
                pltpu.make_async_copy(
                    table_ref.at[idx_v.at[pl.ds(bb * L + off, sz)]],
                    rows_v.at[pl.ds(bb * L + off, sz)],
                    sem,
                )
                for bb in range(chunk_b)
                for off, sz in splits
            ]
            for c in copies:
                c.start()
            for c in copies:
                c.wait()
            for bb in range(chunk_b):
                u0 = u_v[pl.ds(bb * D, 16)]
                u1 = u_v[pl.ds(bb * D + 16, 16)]
                base = bb * L

                def grp_body(gi, csq, base=base, u0=u0, u1=u1, bb=bb):
                    s0, s1 = csq
                    acc, s0, s1 = do_group(base, gi, u0, u1, s0, s1, 16)
                    pred_v[bb, pl.ds(gi * 16, 16)] = acc
                    return (s0, s1)

                sq0, sq1 = lax.fori_loop(0, n_full, grp_body, (sq0, sq1))
                if rem:
                    acc, sq0, sq1 = do_group(base, n_full, u0, u1,
                                             sq0, sq1, rem)
                    pred_v[bb, pl.ds(n_full * 16, 16)] = acc
            pltpu.sync_copy(pred_v.at[:, pl.ds(0, L)],
                            pred_ref.at[pl.ds(b0, chunk_b)])
            return (sq0, sq1)

        z = jnp.zeros((16,), jnp.float32)
        sq0, sq1 = lax.fori_loop(0, n_chunks, chunk_body, (z, z))
        sq_v[...] = sq0 + sq1
        pltpu.sync_copy(sq_v, sq_ref.at[wid])

    return k(item_r, u_flat, table)


_TCB = 2048
_SLAB = _TCB // 4


def _tc_detile_table(tt, n_rows_out):
    Dt, N = tt.shape
    grid = n_rows_out // _TCB

    def body(in_ref, out_ref):
        x = in_ref[...]
        y = jnp.transpose(x)
        out_ref[...] = jnp.concatenate(
            [y[k * _SLAB:(k + 1) * _SLAB, :] for k in range(4)], axis=1)

    return pl.pallas_call(
        body,
        grid=(grid,),
        in_specs=[pl.BlockSpec((Dt, _TCB), lambda i: (0, i))],
        out_specs=pl.BlockSpec((_SLAB, 128), lambda i: (i, 0)),
        out_shape=jax.ShapeDtypeStruct((n_rows_out * Dt // 128, 128),
                                       jnp.float32),
    )(tt)


def _tc_loss(pred, labels, mdsk, u2d, sqp):
    B, L = pred.shape
    D = u2d.shape[1]
    blk = 1024
    grid = B // blk

    def body(pred_ref, lab_ref, msk_ref, u_ref, sq_ref, out_ref, acc_ref):
        i = pl.program_id(0)

        @pl.when(i == 0)
        def _():
            acc_ref[0] = 0.0
            acc_ref[1] = 0.0

        x = pred_ref[...]
        t = lab_ref[...]
        m = msk_ref[...]
        bce = jnp.maximum(x, 0.0) - x * t + jnp.log(1.0 + jnp.exp(-jnp.abs(x)))
        acc_ref[0] += jnp.sum(bce * m)
        acc_ref[1] += jnp.sum(u_ref[...] * u_ref[...])

        @pl.when(i == grid - 1)
        def _():
            gsq = jnp.sum(sq_ref[...])
            out_ref[0, 0] = acc_ref[0] + LAM_U * (
                jnp.sqrt(acc_ref[1]) + jnp.sqrt(gsq))

    out = pl.pallas_call(
        body,
        grid=(grid,),
        in_specs=[
            pl.BlockSpec((blk, L), lambda i: (i, 0)),
            pl.BlockSpec((blk, L), lambda i: (i, 0)),
            pl.BlockSpec((blk, L), lambda i: (i, 0)),
            pl.BlockSpec((blk, D), lambda i: (i, 0)),
            pl.BlockSpec(sqp.shape, lambda i: (0, 0)),
        ],
        out_specs=pl.BlockSpec(memory_space=pltpu.SMEM),
        out_shape=jax.ShapeDtypeStruct((1, 1), jnp.float32),
        scratch_shapes=[pltpu.SMEM((2,), jnp.float32)],
    )(pred, labels, mdsk, u2d, sqp)
    return out[0, 0]


def kernel(user_embedding_update, item, labels, mdsk, item_embeddings):
    B, L = item.shape
    D = user_embedding_update.shape[-1]
    u2d = user_embedding_update.reshape(B, D)
    n_rows = item_embeddings.shape[0]
    n_pad = (n_rows + _TCB - 1) // _TCB * _TCB
    table_lin = _tc_detile_table(
        jnp.transpose(item_embeddings), n_pad).reshape(n_pad, D)
    it = item.astype(jnp.int32)
    it = (it & ~(_TCB - 1)) | ((it & (_SLAB - 1)) << 2) | (
        (it & (_TCB - 1)) >> 9)
    pred, sqp = _sc_gather_dot(
        it.reshape(B * L), u2d.reshape(B * D), table_lin, B, L, D)
    return _tc_loss(pred, labels, mdsk, u2d, sqp)


# speedup vs baseline: 1.1868x; 1.0961x over previous
import functools

import jax
import jax.numpy as jnp
from jax import lax
from jax.experimental import pallas as pl
from jax.experimental.pallas import tpu as pltpu
from jax.experimental.pallas import tpu_sc as plsc

NC = 2
NS = 16
NW = NC * NS
LAM_U = 0.01


def _sc_gather_dot(item_r, u_flat, table, B, L, D):
    b_per_w = B // NW
    chunk_b = 8
    n_chunks = b_per_w // chunk_b
    rows_per_chunk = chunk_b * L
    splits = ((0, 96), (96, 104))
    n_full = L // 16
    rem = L - n_full * 16
    l_pad = (n_full + 1) * 16

    mesh = plsc.VectorSubcoreMesh(core_axis_name="c", subcore_axis_name="s")

    @functools.partial(
        pl.kernel,
        out_type=(
            jax.ShapeDtypeStruct((B, L), jnp.float32),
            jax.ShapeDtypeStruct((NW, 16), jnp.float32),
        ),
        mesh=mesh,
        scratch_types=[
            pltpu.VMEM((rows_per_chunk,), jnp.int32),
            pltpu.VMEM((rows_per_chunk, D), jnp.float32),
            pltpu.VMEM((chunk_b * D,), jnp.float32),
            pltpu.VMEM((chunk_b, l_pad), jnp.float32),
            pltpu.VMEM((16,), jnp.float32),
            pltpu.SemaphoreType.DMA,
        ],
        compiler_params=pltpu.CompilerParams(
            use_tc_tiling_on_sc=False, needs_layout_passes=False),
    )
    def k(item_ref, u_ref, table_ref, pred_ref, sq_ref,
          idx_v, rows_v, u_v, pred_v, sq_v, sem):
        wid = lax.axis_index("s") * NC + lax.axis_index("c")
        b0w = wid * b_per_w
        lane = lax.iota(jnp.int32, 16)

        def do_group(base, gi, u0, u1, s0, s1, nrows):
            acc = jnp.zeros((16,), jnp.float32)
            for r in range(nrows):
                i = base + gi * 16 + r
                r0 = rows_v[i, pl.ds(0, 16)]
                r1 = rows_v[i, pl.ds(16, 16)]
                p = jnp.sum(r0 * u0 + r1 * u1)
                acc = jnp.where(lane == r, p, acc)
                s0 = s0 + r0 * r0
                s1 = s1 + r1 * r1
            return acc, s0, s1

        def chunk_body(ci, carry):
            sq0, sq1 = carry
            b0 = b0w + ci * chunk_b
            pltpu.sync_copy(item_ref.at[pl.ds(b0 * L, rows_per_chunk)], idx_v)
            pltpu.sync_copy(u_ref.at[pl.ds(b0 * D, chunk_b * D)], u_v)
            copies = [
                pltpu.make_async_copy(
                    table_ref.at[idx_v.at[pl.ds(bb * L + off, sz)]],
                    rows_v.at[pl.ds(bb * L + off, sz)],
                    sem,
                )
                for bb in range(chunk_b)
                for off, sz in splits
            ]
            for c in copies:
                c.start()
            for c in copies:
                c.wait()
            for bb in range(chunk_b):
                u0 = u_v[pl.ds(bb * D, 16)]
                u1 = u_v[pl.ds(bb * D + 16, 16)]
                base = bb * L

                def grp_body(gi, csq, base=base, u0=u0, u1=u1, bb=bb):
                    s0, s1 = csq
                    acc, s0, s1 = do_group(base, gi, u0, u1, s0, s1, 16)
                    pred_v[bb, pl.ds(gi * 16, 16)] = acc
                    return (s0, s1)

                sq0, sq1 = lax.fori_loop(0, n_full, grp_body, (sq0, sq1))
                if rem:
                    acc, sq0, sq1 = do_group(base, n_full, u0, u1,
                                             sq0, sq1, rem)
                    pred_v[bb, pl.ds(n_full * 16, 16)] = acc
            pltpu.sync_copy(pred_v.at[:, pl.ds(0, L)],
                            pred_ref.at[pl.ds(b0, chunk_b)])
            return (sq0, sq1)

        z = jnp.zeros((16,), jnp.float32)
        sq0, sq1 = lax.fori_loop(0, n_chunks, chunk_body, (z, z))
        sq_v[...] = sq0 + sq1
        pltpu.sync_copy(sq_v, sq_ref.at[wid])

    return k(item_r, u_flat, table)


_TCB = 2048
_SLAB = _TCB // 4


def _tc_detile_table(tt, n_rows_out):
    Dt, N = tt.shape
    grid = n_rows_out // _TCB

    def body(in_ref, out_ref):
        x = in_ref[...]
        z = jnp.concatenate(
            [x[:, k * _SLAB:(k + 1) * _SLAB] for k in range(4)], axis=0)
        out_ref[...] = jnp.transpose(z)

    return pl.pallas_call(
        body,
        grid=(grid,),
        in_specs=[pl.BlockSpec((Dt, _TCB), lambda i: (0, i))],
        out_specs=pl.BlockSpec((_SLAB, 128), lambda i: (i, 0)),
        out_shape=jax.ShapeDtypeStruct((n_rows_out * Dt // 128, 128),
                                       jnp.float32),
    )(tt)


def _tc_loss(pred, labels, mdsk, u2d, sqp):
    B, L = pred.shape
    D = u2d.shape[1]
    blk = 1024
    grid = B // blk

    def body(pred_ref, lab_ref, msk_ref, u_ref, sq_ref, out_ref, acc_ref):
        i = pl.program_id(0)

        @pl.when(i == 0)
        def _():
            acc_ref[0] = 0.0
            acc_ref[1] = 0.0

        x = pred_ref[...]
        t = lab_ref[...]
        m = msk_ref[...]
        bce = jnp.maximum(x, 0.0) - x * t + jnp.log(1.0 + jnp.exp(-jnp.abs(x)))
        acc_ref[0] += jnp.sum(bce * m)
        acc_ref[1] += jnp.sum(u_ref[...] * u_ref[...])

        @pl.when(i == grid - 1)
        def _():
            gsq = jnp.sum(sq_ref[...])
            out_ref[0, 0] = acc_ref[0] + LAM_U * (
                jnp.sqrt(acc_ref[1]) + jnp.sqrt(gsq))

    out = pl.pallas_call(
        body,
        grid=(grid,),
        in_specs=[
            pl.BlockSpec((blk, L), lambda i: (i, 0)),
            pl.BlockSpec((blk, L), lambda i: (i, 0)),
            pl.BlockSpec((blk, L), lambda i: (i, 0)),
            pl.BlockSpec((blk, D), lambda i: (i, 0)),
            pl.BlockSpec(sqp.shape, lambda i: (0, 0)),
        ],
        out_specs=pl.BlockSpec(memory_space=pltpu.SMEM),
        out_shape=jax.ShapeDtypeStruct((1, 1), jnp.float32),
        scratch_shapes=[pltpu.SMEM((2,), jnp.float32)],
    )(pred, labels, mdsk, u2d, sqp)
    return out[0, 0]


def kernel(user_embedding_update, item, labels, mdsk, item_embeddings):
    B, L = item.shape
    D = user_embedding_update.shape[-1]
    u2d = user_embedding_update.reshape(B, D)
    n_rows = item_embeddings.shape[0]
    n_pad = (n_rows + _TCB - 1) // _TCB * _TCB
    table_lin = _tc_detile_table(
        jnp.transpose(item_embeddings), n_pad).reshape(n_pad, D)
    it = item.astype(jnp.int32)
    it = (it & ~(_TCB - 1)) | ((it & (_SLAB - 1)) << 2) | (
        (it & (_TCB - 1)) >> 9)
    pred, sqp = _sc_gather_dot(
        it.reshape(B * L), u2d.reshape(B * D), table_lin, B, L, D)
    return _tc_loss(pred, labels, mdsk, u2d, sqp)


# speedup vs baseline: 1.3697x; 1.1542x over previous
import functools

import jax
import jax.numpy as jnp
from jax import lax
from jax.experimental import pallas as pl
from jax.experimental.pallas import tpu as pltpu
from jax.experimental.pallas import tpu_sc as plsc

NC = 2
NS = 16
NW = NC * NS
LAM_U = 0.01


def _sc_gather_dot(item_r, u_flat, table, B, L, D):
    b_per_w = B // NW
    chunk_b = 8
    n_chunks = b_per_w // chunk_b
    rows_per_chunk = chunk_b * L
    splits = ((0, 96), (96, 104))
    n_full = L // 16
    rem = L - n_full * 16
    l_pad = (n_full + 1) * 16

    mesh = plsc.VectorSubcoreMesh(core_axis_name="c", subcore_axis_name="s")

    @functools.partial(
        pl.kernel,
        out_type=(
            jax.ShapeDtypeStruct((B, L), jnp.float32),
            jax.ShapeDtypeStruct((NW, 16), jnp.float32),
        ),
        mesh=mesh,
        scratch_types=[
            pltpu.VMEM((rows_per_chunk,), jnp.int32),
            pltpu.VMEM((rows_per_chunk, D), jnp.float32),
            pltpu.VMEM((chunk_b * D,), jnp.float32),
            pltpu.VMEM((chunk_b, l_pad), jnp.float32),
            pltpu.VMEM((16,), jnp.float32),
            pltpu.SemaphoreType.DMA,
        ],
        compiler_params=pltpu.CompilerParams(
            use_tc_tiling_on_sc=False, needs_layout_passes=False),
    )
    def k(item_ref, u_ref, table_ref, pred_ref, sq_ref,
          idx_v, rows_v, u_v, pred_v, sq_v, sem):
        wid = lax.axis_index("s") * NC + lax.axis_index("c")
        b0w = wid * b_per_w
        lane = lax.iota(jnp.int32, 16)

        def do_group(base, gi, u0, u1, s0, s1, nrows):
            acc = jnp.zeros((16,), jnp.float32)
            for r in range(nrows):
                i = base + gi * 16 + r
                r0 = rows_v[i, pl.ds(0, 16)]
                r1 = rows_v[i, pl.ds(16, 16)]
                p = jnp.sum(r0 * u0 + r1 * u1)
                acc = jnp.where(lane == r, p, acc)
                s0 = s0 + r0 * r0
                s1 = s1 + r1 * r1
            return acc, s0, s1

        def chunk_body(ci, carry):
            sq0, sq1 = carry
            b0 = b0w + ci * chunk_b
            pltpu.sync_copy(item_ref.at[pl.ds(b0 * L, rows_per_chunk)], idx_v)
            pltpu.sync_copy(u_ref.at[pl.ds(b0 * D, chunk_b * D)], u_v)
            copies = [
                pltpu.make_async_copy(
                    table_ref.at[idx_v.at[pl.ds(bb * L + off, sz)]],
                    rows_v.at[pl.ds(bb * L + off, sz)],
                    sem,
                )
                for bb in range(chunk_b)
                for off, sz in splits
            ]
            for c in copies:
                c.start()
            for c in copies:
                c.wait()
            for bb in range(chunk_b):
                u0 = u_v[pl.ds(bb * D, 16)]
                u1 = u_v[pl.ds(bb * D + 16, 16)]
                base = bb * L

                def grp_body(gi, csq, base=base, u0=u0, u1=u1, bb=bb):
                    s0, s1 = csq
                    acc, s0, s1 = do_group(base, gi, u0, u1, s0, s1, 16)
                    pred_v[bb, pl.ds(gi * 16, 16)] = acc
                    return (s0, s1)

                sq0, sq1 = lax.fori_loop(0, n_full, grp_body, (sq0, sq1))
                if rem:
                    acc, sq0, sq1 = do_group(base, n_full, u0, u1,
                                             sq0, sq1, rem)
                    pred_v[bb, pl.ds(n_full * 16, 16)] = acc
            pltpu.sync_copy(pred_v.at[:, pl.ds(0, L)],
                            pred_ref.at[pl.ds(b0, chunk_b)])
            return (sq0, sq1)

        z = jnp.zeros((16,), jnp.float32)
        sq0, sq1 = lax.fori_loop(0, n_chunks, chunk_body, (z, z))
        sq_v[...] = sq0 + sq1
        pltpu.sync_copy(sq_v, sq_ref.at[wid])

    return k(item_r, u_flat, table)


_TCB = 4096
_SLAB = _TCB // 4
_SHIFT = _SLAB.bit_length() - 1


def _tc_detile_table(tt, n_rows_out):
    Dt, N = tt.shape
    grid = n_rows_out // _TCB

    def body(in_ref, out_ref):
        x = in_ref[...]
        z = jnp.concatenate(
            [x[:, k * _SLAB:(k + 1) * _SLAB] for k in range(4)], axis=0)
        out_ref[...] = jnp.transpose(z)

    return pl.pallas_call(
        body,
        grid=(grid,),
        in_specs=[pl.BlockSpec((Dt, _TCB), lambda i: (0, i))],
        out_specs=pl.BlockSpec((_SLAB, 128), lambda i: (i, 0)),
        out_shape=jax.ShapeDtypeStruct((n_rows_out * Dt // 128, 128),
                                       jnp.float32),
    )(tt)


def _tc_loss(pred, labels, mdsk, u2d, sqp):
    B, L = pred.shape
    D = u2d.shape[1]
    blk = 1024
    grid = B // blk

    def body(pred_ref, lab_ref, msk_ref, u_ref, sq_ref, out_ref, acc_ref):
        i = pl.program_id(0)

        @pl.when(i == 0)
        def _():
            acc_ref[0] = 0.0
            acc_ref[1] = 0.0

        x = pred_ref[...]
        t = lab_ref[...]
        m = msk_ref[...]
        bce = jnp.maximum(x, 0.0) - x * t + jnp.log(1.0 + jnp.exp(-jnp.abs(x)))
        acc_ref[0] += jnp.sum(bce * m)
        acc_ref[1] += jnp.sum(u_ref[...] * u_ref[...])

        @pl.when(i == grid - 1)
        def _():
            gsq = jnp.sum(sq_ref[...])
            out_ref[0, 0] = acc_ref[0] + LAM_U * (
                jnp.sqrt(acc_ref[1]) + jnp.sqrt(gsq))

    out = pl.pallas_call(
        body,
        grid=(grid,),
        in_specs=[
            pl.BlockSpec((blk, L), lambda i: (i, 0)),
            pl.BlockSpec((blk, L), lambda i: (i, 0)),
            pl.BlockSpec((blk, L), lambda i: (i, 0)),
            pl.BlockSpec((blk, D), lambda i: (i, 0)),
            pl.BlockSpec(sqp.shape, lambda i: (0, 0)),
        ],
        out_specs=pl.BlockSpec(memory_space=pltpu.SMEM),
        out_shape=jax.ShapeDtypeStruct((1, 1), jnp.float32),
        scratch_shapes=[pltpu.SMEM((2,), jnp.float32)],
    )(pred, labels, mdsk, u2d, sqp)
    return out[0, 0]


def kernel(user_embedding_update, item, labels, mdsk, item_embeddings):
    B, L = item.shape
    D = user_embedding_update.shape[-1]
    u2d = user_embedding_update.reshape(B, D)
    n_rows = item_embeddings.shape[0]
    n_pad = (n_rows + _TCB - 1) // _TCB * _TCB
    table_lin = _tc_detile_table(
        jnp.transpose(item_embeddings), n_pad).reshape(n_pad, D)
    it = item.astype(jnp.int32)
    it = (it & ~(_TCB - 1)) | ((it & (_SLAB - 1)) << 2) | (
        (it & (_TCB - 1)) >> _SHIFT)
    pred, sqp = _sc_gather_dot(
        it.reshape(B * L), u2d.reshape(B * D), table_lin, B, L, D)
    return _tc_loss(pred, labels, mdsk, u2d, sqp)


# speedup vs baseline: 1.4706x; 1.0736x over previous
import functools

import jax
import jax.numpy as jnp
from jax import lax
from jax.experimental import pallas as pl
from jax.experimental.pallas import tpu as pltpu
from jax.experimental.pallas import tpu_sc as plsc

NC = 2
NS = 16
NW = NC * NS
LAM_U = 0.01


def _sc_gather_dot(item_r, u_flat, table, B, L, D):
    b_per_w = B // NW
    chunk_b = 8
    n_chunks = b_per_w // chunk_b
    rows_per_chunk = chunk_b * L
    splits = ((0, 96), (96, 104))
    n_full = L // 16
    rem = L - n_full * 16
    l_pad = (n_full + 1) * 16

    mesh = plsc.VectorSubcoreMesh(core_axis_name="c", subcore_axis_name="s")

    @functools.partial(
        pl.kernel,
        out_type=(
            jax.ShapeDtypeStruct((B, L), jnp.float32),
            jax.ShapeDtypeStruct((NW, 16), jnp.float32),
        ),
        mesh=mesh,
        scratch_types=[
            pltpu.VMEM((rows_per_chunk,), jnp.int32),
            pltpu.VMEM((rows_per_chunk, D), jnp.float32),
            pltpu.VMEM((chunk_b * D,), jnp.float32),
            pltpu.VMEM((chunk_b, l_pad), jnp.float32),
            pltpu.VMEM((16,), jnp.float32),
            pltpu.SemaphoreType.DMA,
        ],
        compiler_params=pltpu.CompilerParams(
            use_tc_tiling_on_sc=False, needs_layout_passes=False),
    )
    def k(item_ref, u_ref, table_ref, pred_ref, sq_ref,
          idx_v, rows_v, u_v, pred_v, sq_v, sem):
        wid = lax.axis_index("s") * NC + lax.axis_index("c")
        b0w = wid * b_per_w
        lane = lax.iota(jnp.int32, 16)

        def do_group(base, gi, u0, u1, s0, s1, nrows):
            acc = jnp.zeros((16,), jnp.float32)
            for r in range(nrows):
                i = base + gi * 16 + r
                r0 = rows_v[i, pl.ds(0, 16)]
                r1 = rows_v[i, pl.ds(16, 16)]
                p = jnp.sum(r0 * u0 + r1 * u1)
                acc = jnp.where(lane == r, p, acc)
                s0 = s0 + r0 * r0
                s1 = s1 + r1 * r1
            return acc, s0, s1

        def chunk_body(ci, carry):
            sq0, sq1 = carry
            b0 = b0w + ci * chunk_b
            pltpu.sync_copy(item_ref.at[pl.ds(b0 * L, rows_per_chunk)], idx_v)
            pltpu.sync_copy(u_ref.at[pl.ds(b0 * D, chunk_b * D)], u_v)
            copies = [
                pltpu.make_async_copy(
                    table_ref.at[idx_v.at[pl.ds(bb * L + off, sz)]],
                    rows_v.at[pl.ds(bb * L + off, sz)],
                    sem,
                )
                for bb in range(chunk_b)
                for off, sz in splits
            ]
            for c in copies:
                c.start()
            for c in copies:
                c.wait()
            for bb in range(chunk_b):
                u0 = u_v[pl.ds(bb * D, 16)]
                u1 = u_v[pl.ds(bb * D + 16, 16)]
                base = bb * L

                def grp_body(gi, csq, base=base, u0=u0, u1=u1, bb=bb):
                    s0, s1 = csq
                    acc, s0, s1 = do_group(base, gi, u0, u1, s0, s1, 16)
                    pred_v[bb, pl.ds(gi * 16, 16)] = acc
                    return (s0, s1)

                sq0, sq1 = lax.fori_loop(0, n_full, grp_body, (sq0, sq1))
                if rem:
                    acc, sq0, sq1 = do_group(base, n_full, u0, u1,
                                             sq0, sq1, rem)
                    pred_v[bb, pl.ds(n_full * 16, 16)] = acc
            pltpu.sync_copy(pred_v.at[:, pl.ds(0, L)],
                            pred_ref.at[pl.ds(b0, chunk_b)])
            return (sq0, sq1)

        z = jnp.zeros((16,), jnp.float32)
        sq0, sq1 = lax.fori_loop(0, n_chunks, chunk_body, (z, z))
        sq_v[...] = sq0 + sq1
        pltpu.sync_copy(sq_v, sq_ref.at[wid])

    return k(item_r, u_flat, table)


_TCB = 8192
_SLAB = _TCB // 4
_SHIFT = _SLAB.bit_length() - 1


def _tc_detile_table(tt, n_rows_out):
    Dt, N = tt.shape
    grid = n_rows_out // _TCB

    def body(in_ref, out_ref):
        x = in_ref[...]
        z = jnp.concatenate(
            [x[:, k * _SLAB:(k + 1) * _SLAB] for k in range(4)], axis=0)
        out_ref[...] = jnp.transpose(z)

    return pl.pallas_call(
        body,
        grid=(grid,),
        in_specs=[pl.BlockSpec((Dt, _TCB), lambda i: (0, i))],
        out_specs=pl.BlockSpec((_SLAB, 128), lambda i: (i, 0)),
        out_shape=jax.ShapeDtypeStruct((n_rows_out * Dt // 128, 128),
                                       jnp.float32),
    )(tt)


def _tc_loss(pred, labels, mdsk, u2d, sqp):
    B, L = pred.shape
    D = u2d.shape[1]
    blk = 1024
    grid = B // blk

    def body(pred_ref, lab_ref, msk_ref, u_ref, sq_ref, out_ref, acc_ref):
        i = pl.program_id(0)

        @pl.when(i == 0)
        def _():
            acc_ref[0] = 0.0
            acc_ref[1] = 0.0

        x = pred_ref[...]
        t = lab_ref[...]
        m = msk_ref[...]
        bce = jnp.maximum(x, 0.0) - x * t + jnp.log(1.0 + jnp.exp(-jnp.abs(x)))
        acc_ref[0] += jnp.sum(bce * m)
        acc_ref[1] += jnp.sum(u_ref[...] * u_ref[...])

        @pl.when(i == grid - 1)
        def _():
            gsq = jnp.sum(sq_ref[...])
            out_ref[0, 0] = acc_ref[0] + LAM_U * (
                jnp.sqrt(acc_ref[1]) + jnp.sqrt(gsq))

    out = pl.pallas_call(
        body,
        grid=(grid,),
        in_specs=[
            pl.BlockSpec((blk, L), lambda i: (i, 0)),
            pl.BlockSpec((blk, L), lambda i: (i, 0)),
            pl.BlockSpec((blk, L), lambda i: (i, 0)),
            pl.BlockSpec((blk, D), lambda i: (i, 0)),
            pl.BlockSpec(sqp.shape, lambda i: (0, 0)),
        ],
        out_specs=pl.BlockSpec(memory_space=pltpu.SMEM),
        out_shape=jax.ShapeDtypeStruct((1, 1), jnp.float32),
        scratch_shapes=[pltpu.SMEM((2,), jnp.float32)],
    )(pred, labels, mdsk, u2d, sqp)
    return out[0, 0]


def kernel(user_embedding_update, item, labels, mdsk, item_embeddings):
    B, L = item.shape
    D = user_embedding_update.shape[-1]
    u2d = user_embedding_update.reshape(B, D)
    n_rows = item_embeddings.shape[0]
    n_pad = (n_rows + _TCB - 1) // _TCB * _TCB
    table_lin = _tc_detile_table(
        jnp.transpose(item_embeddings), n_pad).reshape(n_pad, D)
    it = item.astype(jnp.int32)
    it = (it & ~(_TCB - 1)) | ((it & (_SLAB - 1)) << 2) | (
        (it & (_TCB - 1)) >> _SHIFT)
    pred, sqp = _sc_gather_dot(
        it.reshape(B * L), u2d.reshape(B * D), table_lin, B, L, D)
    return _tc_loss(pred, labels, mdsk, u2d, sqp)


# speedup vs baseline: 1.5503x; 1.0542x over previous
import functools

import jax
import jax.numpy as jnp
from jax import lax
from jax.experimental import pallas as pl
from jax.experimental.pallas import tpu as pltpu
from jax.experimental.pallas import tpu_sc as plsc

NC = 2
NS = 16
NW = NC * NS
LAM_U = 0.01


def _sc_gather_dot(item_r, u_flat, table, B, L, D):
    b_per_w = B // NW
    chunk_b = 8
    n_chunks = b_per_w // chunk_b
    rows_per_chunk = chunk_b * L
    splits = ((0, 96), (96, 104))
    n_full = L // 16
    rem = L - n_full * 16
    l_pad = (n_full + 1) * 16

    mesh = plsc.VectorSubcoreMesh(core_axis_name="c", subcore_axis_name="s")

    @functools.partial(
        pl.kernel,
        out_type=(
            jax.ShapeDtypeStruct((B, L), jnp.float32),
            jax.ShapeDtypeStruct((NW, 16), jnp.float32),
        ),
        mesh=mesh,
        scratch_types=[
            pltpu.VMEM((rows_per_chunk,), jnp.int32),
            pltpu.VMEM((rows_per_chunk, D), jnp.float32),
            pltpu.VMEM((chunk_b * D,), jnp.float32),
            pltpu.VMEM((chunk_b, l_pad), jnp.float32),
            pltpu.VMEM((16,), jnp.float32),
            pltpu.SemaphoreType.DMA,
        ],
        compiler_params=pltpu.CompilerParams(
            use_tc_tiling_on_sc=False, needs_layout_passes=False),
    )
    def k(item_ref, u_ref, table_ref, pred_ref, sq_ref,
          idx_v, rows_v, u_v, pred_v, sq_v, sem):
        wid = lax.axis_index("s") * NC + lax.axis_index("c")
        b0w = wid * b_per_w
        lane = lax.iota(jnp.int32, 16)

        def do_group(base, gi, u0, u1, s0, s1, nrows):
            acc = jnp.zeros((16,), jnp.float32)
            for r in range(nrows):
                i = base + gi * 16 + r
                r0 = rows_v[i, pl.ds(0, 16)]
                r1 = rows_v[i, pl.ds(16, 16)]
                p = jnp.sum(r0 * u0 + r1 * u1)
                acc = jnp.where(lane == r, p, acc)
                s0 = s0 + r0 * r0
                s1 = s1 + r1 * r1
            return acc, s0, s1

        def chunk_body(ci, carry):
            sq0, sq1 = carry
            b0 = b0w + ci * chunk_b
            pltpu.sync_copy(item_ref.at[pl.ds(b0 * L, rows_per_chunk)], idx_v)
            pltpu.sync_copy(u_ref.at[pl.ds(b0 * D, chunk_b * D)], u_v)
            copies = [
                pltpu.make_async_copy(
                    table_ref.at[idx_v.at[pl.ds(bb * L + off, sz)]],
                    rows_v.at[pl.ds(bb * L + off, sz)],
                    sem,
                )
                for bb in range(chunk_b)
                for off, sz in splits
            ]
            for c in copies:
                c.start()
            for c in copies:
                c.wait()
            for bb in range(chunk_b):
                u0 = u_v[pl.ds(bb * D, 16)]
                u1 = u_v[pl.ds(bb * D + 16, 16)]
                base = bb * L

                def grp_body(gi, csq, base=base, u0=u0, u1=u1, bb=bb):
                    s0, s1 = csq
                    acc, s0, s1 = do_group(base, gi, u0, u1, s0, s1, 16)
                    pred_v[bb, pl.ds(gi * 16, 16)] = acc
                    return (s0, s1)

                sq0, sq1 = lax.fori_loop(0, n_full, grp_body, (sq0, sq1))
                if rem:
                    acc, sq0, sq1 = do_group(base, n_full, u0, u1,
                                             sq0, sq1, rem)
                    pred_v[bb, pl.ds(n_full * 16, 16)] = acc
            pltpu.sync_copy(pred_v.at[:, pl.ds(0, L)],
                            pred_ref.at[pl.ds(b0, chunk_b)])
            return (sq0, sq1)

        z = jnp.zeros((16,), jnp.float32)
        sq0, sq1 = lax.fori_loop(0, n_chunks, chunk_body, (z, z))
        sq_v[...] = sq0 + sq1
        pltpu.sync_copy(sq_v, sq_ref.at[wid])

    return k(item_r, u_flat, table)


_TCB = 16384
_SLAB = _TCB // 4
_SHIFT = _SLAB.bit_length() - 1


def _tc_detile_table(tt, n_rows_out):
    Dt, N = tt.shape
    grid = n_rows_out // _TCB

    def body(in_ref, out_ref):
        x = in_ref[...]
        z = jnp.concatenate(
            [x[:, k * _SLAB:(k + 1) * _SLAB] for k in range(4)], axis=0)
        out_ref[...] = jnp.transpose(z)

    return pl.pallas_call(
        body,
        grid=(grid,),
        in_specs=[pl.BlockSpec((Dt, _TCB), lambda i: (0, i))],
        out_specs=pl.BlockSpec((_SLAB, 128), lambda i: (i, 0)),
        out_shape=jax.ShapeDtypeStruct((n_rows_out * Dt // 128, 128),
                                       jnp.float32),
    )(tt)


def _tc_loss(pred, labels, mdsk, u2d, sqp):
    B, L = pred.shape
    D = u2d.shape[1]
    blk = 1024
    grid = B // blk

    def body(pred_ref, lab_ref, msk_ref, u_ref, sq_ref, out_ref, acc_ref):
        i = pl.program_id(0)

        @pl.when(i == 0)
        def _():
            acc_ref[0] = 0.0
            acc_ref[1] = 0.0

        x = pred_ref[...]
        t = lab_ref[...]
        m = msk_ref[...]
        bce = jnp.maximum(x, 0.0) - x * t + jnp.log(1.0 + jnp.exp(-jnp.abs(x)))
        acc_ref[0] += jnp.sum(bce * m)
        acc_ref[1] += jnp.sum(u_ref[...] * u_ref[...])

        @pl.when(i == grid - 1)
        def _():
            gsq = jnp.sum(sq_ref[...])
            out_ref[0, 0] = acc_ref[0] + LAM_U * (
                jnp.sqrt(acc_ref[1]) + jnp.sqrt(gsq))

    out = pl.pallas_call(
        body,
        grid=(grid,),
        in_specs=[
            pl.BlockSpec((blk, L), lambda i: (i, 0)),
            pl.BlockSpec((blk, L), lambda i: (i, 0)),
            pl.BlockSpec((blk, L), lambda i: (i, 0)),
            pl.BlockSpec((blk, D), lambda i: (i, 0)),
            pl.BlockSpec(sqp.shape, lambda i: (0, 0)),
        ],
        out_specs=pl.BlockSpec(memory_space=pltpu.SMEM),
        out_shape=jax.ShapeDtypeStruct((1, 1), jnp.float32),
        scratch_shapes=[pltpu.SMEM((2,), jnp.float32)],
    )(pred, labels, mdsk, u2d, sqp)
    return out[0, 0]


def kernel(user_embedding_update, item, labels, mdsk, item_embeddings):
    B, L = item.shape
    D = user_embedding_update.shape[-1]
    u2d = user_embedding_update.reshape(B, D)
    n_rows = item_embeddings.shape[0]
    n_pad = (n_rows + _TCB - 1) // _TCB * _TCB
    table_lin = _tc_detile_table(
        jnp.transpose(item_embeddings), n_pad).reshape(n_pad, D)
    it = item.astype(jnp.int32)
    it = (it & ~(_TCB - 1)) | ((it & (_SLAB - 1)) << 2) | (
        (it & (_TCB - 1)) >> _SHIFT)
    pred, sqp = _sc_gather_dot(
        it.reshape(B * L), u2d.reshape(B * D), table_lin, B, L, D)
    return _tc_loss(pred, labels, mdsk, u2d, sqp)


# speedup vs baseline: 1.5808x; 1.0197x over previous
import functools

import jax
import jax.numpy as jnp
from jax import lax
from jax.experimental import pallas as pl
from jax.experimental.pallas import tpu as pltpu
from jax.experimental.pallas import tpu_sc as plsc

NC = 2
NS = 16
NW = NC * NS
LAM_U = 0.01


def _sc_gather_dot(item_r, u_flat, table, B, L, D):
    b_per_w = B // NW
    chunk_b = 8
    n_chunks = b_per_w // chunk_b
    rows_per_chunk = chunk_b * L
    splits = ((0, 96), (96, 104))
    n_full = L // 16
    rem = L - n_full * 16
    l_pad = (n_full + 1) * 16

    mesh = plsc.VectorSubcoreMesh(core_axis_name="c", subcore_axis_name="s")

    @functools.partial(
        pl.kernel,
        out_type=(
            jax.ShapeDtypeStruct((B, L), jnp.float32),
            jax.ShapeDtypeStruct((NW, 16), jnp.float32),
        ),
        mesh=mesh,
        scratch_types=[
            pltpu.VMEM((rows_per_chunk,), jnp.int32),
            pltpu.VMEM((rows_per_chunk, D), jnp.float32),
            pltpu.VMEM((chunk_b * D,), jnp.float32),
            pltpu.VMEM((chunk_b, l_pad), jnp.float32),
            pltpu.VMEM((16,), jnp.float32),
            pltpu.SemaphoreType.DMA,
        ],
        compiler_params=pltpu.CompilerParams(
            use_tc_tiling_on_sc=False, needs_layout_passes=False),
    )
    def k(item_ref, u_ref, table_ref, pred_ref, sq_ref,
          idx_v, rows_v, u_v, pred_v, sq_v, sem):
        wid = lax.axis_index("s") * NC + lax.axis_index("c")
        b0w = wid * b_per_w
        lane = lax.iota(jnp.int32, 16)

        def do_group(base, gi, u0, u1, s0, s1, nrows):
            acc = jnp.zeros((16,), jnp.float32)
            for r in range(nrows):
                i = base + gi * 16 + r
                r0 = rows_v[i, pl.ds(0, 16)]
                r1 = rows_v[i, pl.ds(16, 16)]
                p = jnp.sum(r0 * u0 + r1 * u1)
                acc = jnp.where(lane == r, p, acc)
                s0 = s0 + r0 * r0
                s1 = s1 + r1 * r1
            return acc, s0, s1

        def chunk_body(ci, carry):
            sq0, sq1 = carry
            b0 = b0w + ci * chunk_b
            pltpu.sync_copy(item_ref.at[pl.ds(b0 * L, rows_per_chunk)], idx_v)
            pltpu.sync_copy(u_ref.at[pl.ds(b0 * D, chunk_b * D)], u_v)
            copies = [
                pltpu.make_async_copy(
                    table_ref.at[idx_v.at[pl.ds(bb * L + off, sz)]],
                    rows_v.at[pl.ds(bb * L + off, sz)],
                    sem,
                )
                for bb in range(chunk_b)
                for off, sz in splits
            ]
            for c in copies:
                c.start()
            for c in copies:
                c.wait()
            for bb in range(chunk_b):
                u0 = u_v[pl.ds(bb * D, 16)]
                u1 = u_v[pl.ds(bb * D + 16, 16)]
                base = bb * L

                def grp_body(gi, csq, base=base, u0=u0, u1=u1, bb=bb):
                    s0, s1 = csq
                    acc, s0, s1 = do_group(base, gi, u0, u1, s0, s1, 16)
                    pred_v[bb, pl.ds(gi * 16, 16)] = acc
                    return (s0, s1)

                sq0, sq1 = lax.fori_loop(0, n_full, grp_body, (sq0, sq1))
                if rem:
                    acc, sq0, sq1 = do_group(base, n_full, u0, u1,
                                             sq0, sq1, rem)
                    pred_v[bb, pl.ds(n_full * 16, 16)] = acc
            pltpu.sync_copy(pred_v.at[:, pl.ds(0, L)],
                            pred_ref.at[pl.ds(b0, chunk_b)])
            return (sq0, sq1)

        z = jnp.zeros((16,), jnp.float32)
        sq0, sq1 = lax.fori_loop(0, n_chunks, chunk_body, (z, z))
        sq_v[...] = sq0 + sq1
        pltpu.sync_copy(sq_v, sq_ref.at[wid])

    return k(item_r, u_flat, table)


_TCB = 32768
_SLAB = _TCB // 4
_SHIFT = _SLAB.bit_length() - 1


def _tc_detile_table(tt, n_rows_out):
    Dt, N = tt.shape
    grid = n_rows_out // _TCB

    def body(in_ref, out_ref):
        x = in_ref[...]
        z = jnp.concatenate(
            [x[:, k * _SLAB:(k + 1) * _SLAB] for k in range(4)], axis=0)
        out_ref[...] = jnp.transpose(z)

    return pl.pallas_call(
        body,
        grid=(grid,),
        in_specs=[pl.BlockSpec((Dt, _TCB), lambda i: (0, i))],
        out_specs=pl.BlockSpec((_SLAB, 128), lambda i: (i, 0)),
        out_shape=jax.ShapeDtypeStruct((n_rows_out * Dt // 128, 128),
                                       jnp.float32),
    )(tt)


def _tc_loss(pred, labels, mdsk, u2d, sqp):
    B, L = pred.shape
    D = u2d.shape[1]
    blk = 1024
    grid = B // blk

    def body(pred_ref, lab_ref, msk_ref, u_ref, sq_ref, out_ref, acc_ref):
        i = pl.program_id(0)

        @pl.when(i == 0)
        def _():
            acc_ref[0] = 0.0
            acc_ref[1] = 0.0

        x = pred_ref[...]
        t = lab_ref[...]
        m = msk_ref[...]
        bce = jnp.maximum(x, 0.0) - x * t + jnp.log(1.0 + jnp.exp(-jnp.abs(x)))
        acc_ref[0] += jnp.sum(bce * m)
        acc_ref[1] += jnp.sum(u_ref[...] * u_ref[...])

        @pl.when(i == grid - 1)
        def _():
            gsq = jnp.sum(sq_ref[...])
            out_ref[0, 0] = acc_ref[0] + LAM_U * (
                jnp.sqrt(acc_ref[1]) + jnp.sqrt(gsq))

    out = pl.pallas_call(
        body,
        grid=(grid,),
        in_specs=[
            pl.BlockSpec((blk, L), lambda i: (i, 0)),
            pl.BlockSpec((blk, L), lambda i: (i, 0)),
            pl.BlockSpec((blk, L), lambda i: (i, 0)),
            pl.BlockSpec((blk, D), lambda i: (i, 0)),
            pl.BlockSpec(sqp.shape, lambda i: (0, 0)),
        ],
        out_specs=pl.BlockSpec(memory_space=pltpu.SMEM),
        out_shape=jax.ShapeDtypeStruct((1, 1), jnp.float32),
        scratch_shapes=[pltpu.SMEM((2,), jnp.float32)],
    )(pred, labels, mdsk, u2d, sqp)
    return out[0, 0]


def kernel(user_embedding_update, item, labels, mdsk, item_embeddings):
    B, L = item.shape
    D = user_embedding_update.shape[-1]
    u2d = user_embedding_update.reshape(B, D)
    n_rows = item_embeddings.shape[0]
    n_pad = (n_rows + _TCB - 1) // _TCB * _TCB
    table_lin = _tc_detile_table(
        jnp.transpose(item_embeddings), n_pad).reshape(n_pad, D)
    it = item.astype(jnp.int32)
    it = (it & ~(_TCB - 1)) | ((it & (_SLAB - 1)) << 2) | (
        (it & (_TCB - 1)) >> _SHIFT)
    pred, sqp = _sc_gather_dot(
        it.reshape(B * L), u2d.reshape(B * D), table_lin, B, L, D)
    return _tc_loss(pred, labels, mdsk, u2d, sqp)
